# Initial kernel scaffold; baseline (speedup 1.0000x reference)
import jax
import jax.numpy as jnp
from jax.experimental import pallas as pl


def kernel(input_ids, table1, table2, table3, W):
    raise NotImplementedError("write your pallas kernel here")


# speedup vs baseline: 8.5131x; 8.5131x over previous
import functools

import jax
import jax.numpy as jnp
from jax import lax
from jax.experimental import pallas as pl
from jax.experimental.pallas import tpu as pltpu
from jax.experimental.pallas import tpu_sc as plsc

TABLE_MOD = 1000000
DIM = 32
HIDDEN = 128

_HASH_C = (8191, 104729, 2097593)
_CP = tuple(c % TABLE_MOD for c in _HASH_C)
_K = tuple((1024 * c) % TABLE_MOD for c in _HASH_C)

_NC = 2
_NS = 16
_NW = _NC * _NS
_CHUNK = 128
_LANES = 16


def _hash16(p, c, K, Cp):
    p1 = p >> 10
    p0 = p & 1023
    v = p1 * K + p0 * Cp + c
    m = TABLE_MOD * 128
    for _ in range(8):
        v = jnp.where(v >= m, v - m, v)
        m //= 2
    return v


def _sc_gather(ids_flat, prev_flat, table1, table2, table3):
    n = ids_flat.shape[0]
    npw = n // _NW
    nchunks = npw // _CHUNK
    mesh = plsc.VectorSubcoreMesh(core_axis_name="c", subcore_axis_name="s")

    @functools.partial(
        pl.kernel,
        out_type=(
            jax.ShapeDtypeStruct((n, DIM), jnp.float32),
            jax.ShapeDtypeStruct((n, DIM), jnp.float32),
            jax.ShapeDtypeStruct((n, DIM), jnp.float32),
        ),
        mesh=mesh,
        compiler_params=pltpu.CompilerParams(use_tc_tiling_on_sc=False),
        scratch_types=[
            pltpu.VMEM((npw,), jnp.int32),
            pltpu.VMEM((npw,), jnp.int32),
            pltpu.VMEM((_CHUNK,), jnp.int32),
            pltpu.VMEM((_CHUNK,), jnp.int32),
            pltpu.VMEM((_CHUNK,), jnp.int32),
            pltpu.VMEM((_CHUNK, DIM), jnp.float32),
            pltpu.VMEM((_CHUNK, DIM), jnp.float32),
            pltpu.VMEM((_CHUNK, DIM), jnp.float32),
            pltpu.SemaphoreType.DMA,
        ],
    )
    def k(ids_hbm, prev_hbm, t1, t2, t3, e1, e2, e3,
          ids_v, prev_v, i1, i2, i3, r1, r2, r3, sem):
        wid = lax.axis_index("s") * _NC + lax.axis_index("c")
        base = wid * npw
        pltpu.sync_copy(ids_hbm.at[pl.ds(base, npw)], ids_v)
        pltpu.sync_copy(prev_hbm.at[pl.ds(base, npw)], prev_v)

        def chunk_body(cc, carry):
            off = cc * _CHUNK
            for j in range(_CHUNK // _LANES):
                p = prev_v[pl.ds(off + j * _LANES, _LANES)]
                c = ids_v[pl.ds(off + j * _LANES, _LANES)]
                i1[pl.ds(j * _LANES, _LANES)] = _hash16(p, c, _K[0], _CP[0])
                i2[pl.ds(j * _LANES, _LANES)] = _hash16(p, c, _K[1], _CP[1])
                i3[pl.ds(j * _LANES, _LANES)] = _hash16(p, c, _K[2], _CP[2])
            h1 = pltpu.async_copy(t1.at[i1], r1, sem)
            h2 = pltpu.async_copy(t2.at[i2], r2, sem)
            h3 = pltpu.async_copy(t3.at[i3], r3, sem)
            h1.wait()
            pltpu.sync_copy(r1, e1.at[pl.ds(base + off, _CHUNK)])
            h2.wait()
            pltpu.sync_copy(r2, e2.at[pl.ds(base + off, _CHUNK)])
            h3.wait()
            pltpu.sync_copy(r3, e3.at[pl.ds(base + off, _CHUNK)])
            return carry

        lax.fori_loop(jnp.int32(0), jnp.int32(nchunks), chunk_body,
                      jnp.int32(0))

    return k(ids_flat, prev_flat, table1, table2, table3)


def _tc_project(e1, e2, e3, W, blk=2048):
    n = e1.shape[0]

    def body(e1_ref, e2_ref, e3_ref, w_ref, o_ref):
        w = w_ref[...]
        dn = (((1,), (1,)), ((), ()))
        acc = lax.dot_general(e1_ref[...], w[:, 0:DIM], dn,
                              precision=lax.Precision.HIGHEST,
                              preferred_element_type=jnp.float32)
        acc += lax.dot_general(e2_ref[...], w[:, DIM:2 * DIM], dn,
                               precision=lax.Precision.HIGHEST,
                               preferred_element_type=jnp.float32)
        acc += lax.dot_general(e3_ref[...], w[:, 2 * DIM:3 * DIM], dn,
                               precision=lax.Precision.HIGHEST,
                               preferred_element_type=jnp.float32)
        o_ref[...] = acc

    return pl.pallas_call(
        body,
        grid=(n // blk,),
        in_specs=[
            pl.BlockSpec((blk, DIM), lambda i: (i, jnp.int32(0))),
            pl.BlockSpec((blk, DIM), lambda i: (i, jnp.int32(0))),
            pl.BlockSpec((blk, DIM), lambda i: (i, jnp.int32(0))),
            pl.BlockSpec((HIDDEN, 3 * DIM),
                         lambda i: (jnp.int32(0), jnp.int32(0))),
        ],
        out_specs=pl.BlockSpec((blk, HIDDEN), lambda i: (i, jnp.int32(0))),
        out_shape=jax.ShapeDtypeStruct((n, HIDDEN), jnp.float32),
    )(e1, e2, e3, W)


def kernel(input_ids, table1, table2, table3, W):
    b, t = input_ids.shape
    ids32 = input_ids.astype(jnp.int32)
    prev32 = jnp.concatenate(
        [jnp.zeros((b, 1), jnp.int32), ids32[:, :-1]], axis=1)
    ids_flat = ids32.reshape(-1)
    prev_flat = prev32.reshape(-1)
    e1, e2, e3 = _sc_gather(ids_flat, prev_flat, table1, table2, table3)
    out = _tc_project(e1, e2, e3, W)
    return out.reshape(b, t, HIDDEN)
